# baseline (speedup 1.0000x reference)
import functools

import jax
import jax.numpy as jnp
from jax.experimental import pallas as pl
from jax.experimental.pallas import tpu as pltpu

N = 8192
BLOCK_C = 2048


def _nn_kernel(bn2_ref, at_ref, out_ref, acc_ref):
    step = pl.program_id(0)

    bn2 = bn2_ref[...]
    at = at_ref[...]
    b2 = 0.25 * jnp.sum(bn2 * bn2, axis=1, keepdims=True)
    h1 = b2.astype(jnp.bfloat16).astype(jnp.float32)
    r1 = b2 - h1
    h2 = r1.astype(jnp.bfloat16).astype(jnp.float32)
    h3 = (r1 - h2).astype(jnp.bfloat16).astype(jnp.float32)
    z = jnp.zeros_like(h1)
    op = jnp.concatenate([bn2[:, 0:3], h1, h2, h3, z, z], axis=1)
    opb = op.astype(jnp.bfloat16)
    atb = at.astype(jnp.bfloat16)
    u = jnp.dot(opb, atb, preferred_element_type=jnp.float32)
    m = u
    size = BLOCK_C
    while size > 8:
        half = size // 2
        m = jnp.minimum(m[0:half, :], m[half:size, :])
        size = half

    @pl.when(step == 0)
    def _init():
        acc_ref[...] = m

    @pl.when(step != 0)
    def _acc():
        acc_ref[...] = jnp.minimum(acc_ref[...], m)

    @pl.when(step == pl.num_programs(0) - 1)
    def _fin():
        a3 = at[0:3, :]
        a2 = jnp.sum(a3 * a3, axis=0, keepdims=True)
        mfull = jnp.min(acc_ref[...], axis=0, keepdims=True)
        dist = jnp.maximum(a2 + mfull, 0.0)
        mask = dist <= 2.0
        s = jnp.sum(jnp.where(mask, dist, 0.0))
        c = jnp.sum(mask.astype(jnp.float32))
        out_ref[...] = jnp.reshape(s / jnp.maximum(c, 1.0), (1, 1))


@jax.jit
def _nn(bn2, at):
    out = pl.pallas_call(
        _nn_kernel,
        grid=(N // BLOCK_C,),
        in_specs=[
            pl.BlockSpec((BLOCK_C, 8), lambda i: (i, 0)),
            pl.BlockSpec((8, N), lambda i: (0, 0)),
        ],
        out_specs=pl.BlockSpec((1, 1), lambda i: (0, 0)),
        out_shape=jax.ShapeDtypeStruct((1, 1), jnp.float32),
        scratch_shapes=[
            pltpu.VMEM((8, N), jnp.float32),
        ],
    )(bn2, at)
    return out[0, 0]


def kernel(input0, input1):
    bn2 = jnp.zeros((N, 8), jnp.float32).at[:, :3].set(-2.0 * input1)
    at = jnp.zeros((8, N), jnp.float32).at[:3, :].set(input0.T)
    at = at.at[3:6, :].set(1.0)
    return _nn(bn2, at)


# speedup vs baseline: 1.0236x; 1.0236x over previous
import functools

import jax
import jax.numpy as jnp
from jax.experimental import pallas as pl
from jax.experimental.pallas import tpu as pltpu

N = 8192
BLOCK_C = 2048


def _nn_kernel(bn2_ref, at_ref, out_ref, acc_ref):
    step = pl.program_id(0)

    bn2 = bn2_ref[...]
    at = at_ref[...]
    b2 = 0.25 * jnp.sum(bn2 * bn2, axis=1, keepdims=True)
    ut = jnp.dot(bn2, at, preferred_element_type=jnp.float32)
    u = ut + b2
    m = u
    size = BLOCK_C
    while size > 8:
        half = size // 2
        m = jnp.minimum(m[0:half, :], m[half:size, :])
        size = half

    @pl.when(step == 0)
    def _init():
        acc_ref[...] = m

    @pl.when(step != 0)
    def _acc():
        acc_ref[...] = jnp.minimum(acc_ref[...], m)

    @pl.when(step == pl.num_programs(0) - 1)
    def _fin():
        a2 = jnp.sum(at * at, axis=0, keepdims=True)
        mfull = jnp.min(acc_ref[...], axis=0, keepdims=True)
        dist = jnp.maximum(a2 + mfull, 0.0)
        mask = dist <= 2.0
        s = jnp.sum(jnp.where(mask, dist, 0.0))
        c = jnp.sum(mask.astype(jnp.float32))
        out_ref[...] = jnp.reshape(s / jnp.maximum(c, 1.0), (1, 1))


@jax.jit
def _nn(bn2, at):
    out = pl.pallas_call(
        _nn_kernel,
        grid=(N // BLOCK_C,),
        in_specs=[
            pl.BlockSpec((BLOCK_C, 8), lambda i: (i, 0)),
            pl.BlockSpec((8, N), lambda i: (0, 0)),
        ],
        out_specs=pl.BlockSpec((1, 1), lambda i: (0, 0)),
        out_shape=jax.ShapeDtypeStruct((1, 1), jnp.float32),
        scratch_shapes=[
            pltpu.VMEM((8, N), jnp.float32),
        ],
    )(bn2, at)
    return out[0, 0]


def kernel(input0, input1):
    bn2 = jnp.zeros((N, 8), jnp.float32).at[:, :3].set(-2.0 * input1)
    at = jnp.zeros((8, N), jnp.float32).at[:3, :].set(input0.T)
    return _nn(bn2, at)
